# Initial kernel scaffold; baseline (speedup 1.0000x reference)
import jax
import jax.numpy as jnp
from jax.experimental import pallas as pl


def kernel(inputs, embedding):
    raise NotImplementedError("write your pallas kernel here")


# speedup vs baseline: 1.0217x; 1.0217x over previous
import functools

import jax
import jax.numpy as jnp
from jax import lax
from jax.experimental import pallas as pl
from jax.experimental.pallas import tpu as pltpu
from jax.experimental.pallas import tpu_sc as plsc

_FEATURES = 32
_TOTAL = 16384 * 50
_NC, _NS = 2, 16
_NW = _NC * _NS
_PER_W = _TOTAL // _NW
_GROUP = 128
_NGROUP = _PER_W // _GROUP

_mesh = plsc.VectorSubcoreMesh(core_axis_name="c", subcore_axis_name="s")


@functools.partial(
    pl.kernel,
    mesh=_mesh,
    out_type=jax.ShapeDtypeStruct((_TOTAL, _FEATURES), jnp.float32),
    scratch_types=[
        pltpu.VMEM((_NGROUP, _GROUP), jnp.int32),
        pltpu.VMEM((_GROUP, _FEATURES), jnp.float32),
        pltpu.SemaphoreType.DMA,
    ],
    compiler_params=pltpu.CompilerParams(use_tc_tiling_on_sc=False),
)
def _embed_gather(idx_hbm, table_hbm, out_hbm, idx_v, rows_v, sem):
    wid = lax.axis_index("s") * _NC + lax.axis_index("c")
    base = wid * _PER_W
    pltpu.sync_copy(idx_hbm.at[wid], idx_v)

    def body(g, carry):
        pltpu.async_copy(table_hbm.at[idx_v.at[g]], rows_v, sem).wait()
        pltpu.sync_copy(rows_v, out_hbm.at[pl.ds(base + g * _GROUP, _GROUP)])
        return carry

    lax.fori_loop(0, _NGROUP, body, 0)


def kernel(inputs, embedding):
    idx = inputs.reshape(_NW, _NGROUP, _GROUP)
    out = _embed_gather(idx, embedding)
    return out.reshape(inputs.shape + (_FEATURES,))


# speedup vs baseline: 1.1103x; 1.0867x over previous
import functools

import jax
import jax.numpy as jnp
from jax import lax
from jax.experimental import pallas as pl
from jax.experimental.pallas import tpu as pltpu
from jax.experimental.pallas import tpu_sc as plsc

_FEATURES = 32
_TOTAL = 16384 * 50
_NC, _NS = 2, 16
_NW = _NC * _NS
_PER_W = _TOTAL // _NW
_GROUP = 128
_NGROUP = _PER_W // _GROUP
_BK = 10
_BLK_ROWS = _BK * _GROUP
_NBLK = _NGROUP // _BK

_mesh = plsc.VectorSubcoreMesh(core_axis_name="c", subcore_axis_name="s")


@functools.partial(
    pl.kernel,
    mesh=_mesh,
    out_type=jax.ShapeDtypeStruct((_TOTAL, _FEATURES), jnp.float32),
    scratch_types=[
        pltpu.VMEM((_NGROUP, _GROUP), jnp.int32),
        pltpu.VMEM((_BLK_ROWS, _FEATURES), jnp.float32),
        pltpu.VMEM((_BLK_ROWS, _FEATURES), jnp.float32),
        pltpu.SemaphoreType.DMA,
        pltpu.SemaphoreType.DMA,
        pltpu.SemaphoreType.DMA,
        pltpu.SemaphoreType.DMA,
    ],
    compiler_params=pltpu.CompilerParams(use_tc_tiling_on_sc=False),
)
def _embed_gather(idx_hbm, table_hbm, out_hbm, idx_v, rows0, rows1,
                  semg0, semg1, sems0, sems1):
    wid = lax.axis_index("s") * _NC + lax.axis_index("c")
    base = wid * _PER_W
    pltpu.sync_copy(idx_hbm.at[wid], idx_v)

    def issue(blk, rows_b, semg_b):
        for j in range(_BK):
            pltpu.async_copy(
                table_hbm.at[idx_v.at[blk * _BK + j]],
                rows_b.at[pl.ds(j * _GROUP, _GROUP)],
                semg_b,
            )

    def drain_gathers(rows_b, semg_b):
        pltpu.make_async_copy(
            table_hbm.at[pl.ds(0, _BLK_ROWS)], rows_b, semg_b).wait()

    def store(blk, rows_b, sems_b):
        pltpu.async_copy(
            rows_b, out_hbm.at[pl.ds(base + blk * _BLK_ROWS, _BLK_ROWS)],
            sems_b)

    def drain_store(rows_b, sems_b):
        pltpu.make_async_copy(
            rows_b, out_hbm.at[pl.ds(base, _BLK_ROWS)], sems_b).wait()

    issue(0, rows0, semg0)

    def body(i, carry):
        blk0 = 2 * i
        blk1 = blk0 + 1

        @pl.when(i > 0)
        def _():
            drain_store(rows1, sems1)

        issue(blk1, rows1, semg1)
        drain_gathers(rows0, semg0)
        store(blk0, rows0, sems0)

        @pl.when(i < _NBLK // 2 - 1)
        def _():
            drain_store(rows0, sems0)
            issue(blk0 + 2, rows0, semg0)

        drain_gathers(rows1, semg1)
        store(blk1, rows1, sems1)
        return carry

    lax.fori_loop(0, _NBLK // 2, body, 0)
    drain_store(rows0, sems0)
    drain_store(rows1, sems1)


def kernel(inputs, embedding):
    idx = inputs.reshape(_NW, _NGROUP, _GROUP)
    out = _embed_gather(idx, embedding)
    return out.reshape(inputs.shape + (_FEATURES,))


# speedup vs baseline: 1.5153x; 1.3647x over previous
import functools

import jax
import jax.numpy as jnp
from jax import lax
from jax.experimental import pallas as pl
from jax.experimental.pallas import tpu as pltpu
from jax.experimental.pallas import tpu_sc as plsc

_B = 16384
_S = 50
_F = 32
_NC, _NS = 2, 16
_NW = _NC * _NS
_QB = 1024
_NQ = _B // _QB
_NJOB = _S * _NQ
_PER_W = _NJOB // _NW
_GROUP = 128

_mesh = plsc.VectorSubcoreMesh(core_axis_name="c", subcore_axis_name="s")


@functools.partial(
    pl.kernel,
    mesh=_mesh,
    out_type=jax.ShapeDtypeStruct((_S, _F // 8, _B // 128, 8, 128),
                                  jnp.float32),
    scratch_types=[
        pltpu.VMEM((_QB,), jnp.int32),
        pltpu.VMEM((_QB, _F), jnp.float32),
        pltpu.VMEM((_F // 8, _QB // 128, 8, 128), jnp.float32),
        pltpu.SemaphoreType.DMA,
    ],
    compiler_params=pltpu.CompilerParams(
        use_tc_tiling_on_sc=False, needs_layout_passes=False),
)
def _embed_gather(idx_hbm, table_hbm, out_hbm, idx_v, rows_v, outt_v, semg):
    wid = lax.axis_index("s") * _NC + lax.axis_index("c")

    def job(t, carry):
        jg = wid * _PER_W + t
        s = jg // _NQ
        q = jg % _NQ

        pltpu.sync_copy(idx_hbm.at[s, pl.ds(q * _QB, _QB)], idx_v)

        for r in range(_QB // _GROUP):
            pltpu.async_copy(
                table_hbm.at[idx_v.at[pl.ds(r * _GROUP, _GROUP)]],
                rows_v.at[pl.ds(r * _GROUP, _GROUP), :],
                semg,
            )
        pltpu.make_async_copy(
            table_hbm.at[pl.ds(0, _QB)], rows_v, semg).wait()

        def tgroup(g, carry2):
            r0 = g * 16
            bc = g // 8
            row_ids = r0 + lax.iota(jnp.int32, 16)
            for f in range(_F):
                vals = plsc.load_gather(
                    rows_v, [row_ids, jnp.full((16,), f, jnp.int32)])
                outt_v[f // 8, bc, f % 8, pl.ds((g % 8) * 16, 16)] = vals
            return carry2

        lax.fori_loop(0, _QB // 16, tgroup, 0)

        for fb in range(_F // 8):
            pltpu.sync_copy(
                outt_v.at[fb],
                out_hbm.at[s, fb, pl.ds(q * (_QB // 128), _QB // 128)],
            )
        return carry

    lax.fori_loop(0, _PER_W, job, 0)


def kernel(inputs, embedding):
    out5 = _embed_gather(inputs.T, embedding)
    return out5.transpose((2, 4, 0, 1, 3)).reshape(_B, _S, _F)


# speedup vs baseline: 1.7731x; 1.1702x over previous
import functools

import jax
import jax.numpy as jnp
from jax import lax
from jax.experimental import pallas as pl
from jax.experimental.pallas import tpu as pltpu
from jax.experimental.pallas import tpu_sc as plsc

_B = 16384
_S = 50
_F = 32
_NC, _NS = 2, 16
_NW = _NC * _NS
_QB = 1024
_NQ = _B // _QB
_NJOB = _S * _NQ
_PER_W = _NJOB // _NW
_GROUP = 128

_mesh = plsc.VectorSubcoreMesh(core_axis_name="c", subcore_axis_name="s")


@functools.partial(
    pl.kernel,
    mesh=_mesh,
    out_type=jax.ShapeDtypeStruct((_S, _F // 8, _B // 128, 8, 128),
                                  jnp.float32),
    scratch_types=[
        pltpu.VMEM((_QB,), jnp.int32),
        pltpu.VMEM((_QB, _F), jnp.float32),
        pltpu.VMEM((_F // 8, _QB // 128, 8, 128), jnp.float32),
        pltpu.SemaphoreType.DMA,
    ],
    compiler_params=pltpu.CompilerParams(
        use_tc_tiling_on_sc=False, needs_layout_passes=False),
)
def _embed_gather(idx_hbm, table_hbm, out_hbm, idx_v, rows_v, outt_v, semg):
    wid = lax.axis_index("s") * _NC + lax.axis_index("c")

    def job(t, carry):
        jg = wid * _PER_W + t
        s = jg // _NQ
        q = jg % _NQ

        pltpu.sync_copy(idx_hbm.at[s, pl.ds(q * _QB, _QB)], idx_v)

        for r in range(_QB // _GROUP):
            pltpu.async_copy(
                table_hbm.at[idx_v.at[pl.ds(r * _GROUP, _GROUP)]],
                rows_v.at[pl.ds(r * _GROUP, _GROUP), :],
                semg,
            )
        pltpu.make_async_copy(
            table_hbm.at[pl.ds(0, _QB)], rows_v, semg).wait()

        @plsc.parallel_loop(0, _QB // 16, unroll=4)
        def tgroup(g):
            r0 = g * 16
            bc = g // 8
            row_ids = r0 + lax.iota(jnp.int32, 16)
            for f in range(_F):
                vals = plsc.load_gather(
                    rows_v, [row_ids, jnp.full((16,), f, jnp.int32)])
                outt_v[f // 8, bc, f % 8, pl.ds((g % 8) * 16, 16)] = vals

        for fb in range(_F // 8):
            pltpu.sync_copy(
                outt_v.at[fb],
                out_hbm.at[s, fb, pl.ds(q * (_QB // 128), _QB // 128)],
            )
        return carry

    lax.fori_loop(0, _PER_W, job, 0)


def kernel(inputs, embedding):
    out5 = _embed_gather(inputs.T, embedding)
    return out5.transpose((2, 4, 0, 1, 3)).reshape(_B, _S, _F)


# speedup vs baseline: 1.9251x; 1.0857x over previous
import functools

import jax
import jax.numpy as jnp
from jax import lax
from jax.experimental import pallas as pl
from jax.experimental.pallas import tpu as pltpu
from jax.experimental.pallas import tpu_sc as plsc

_B = 16384
_S = 50
_F = 32
_NC, _NS = 2, 16
_NW = _NC * _NS
_QB = 1024
_NQ = _B // _QB
_NJOB = _S * _NQ
_PER_W = _NJOB // _NW
_GROUP = 128

_mesh = plsc.VectorSubcoreMesh(core_axis_name="c", subcore_axis_name="s")


@functools.partial(
    pl.kernel,
    mesh=_mesh,
    out_type=jax.ShapeDtypeStruct((_S, _F // 8, _B // 128, 8, 128),
                                  jnp.float32),
    scratch_types=[
        pltpu.VMEM((_QB,), jnp.int32),
        pltpu.VMEM((_QB,), jnp.int32),
        pltpu.VMEM((_QB, _F), jnp.float32),
        pltpu.VMEM((_QB, _F), jnp.float32),
        pltpu.VMEM((_F // 8, _QB // 128, 8, 128), jnp.float32),
        pltpu.SemaphoreType.DMA,
        pltpu.SemaphoreType.DMA,
        pltpu.SemaphoreType.DMA,
    ],
    compiler_params=pltpu.CompilerParams(
        use_tc_tiling_on_sc=False, needs_layout_passes=False),
)
def _embed_gather(idx_hbm, table_hbm, out_hbm, idx0, idx1, rows0, rows1,
                  outt_v, semg0, semg1, sems):
    wid = lax.axis_index("s") * _NC + lax.axis_index("c")
    jbase = wid * _PER_W

    def fetch(t, idx_v, rows_v, semg):
        jg = jbase + t
        s = jg // _NQ
        q = jg % _NQ
        pltpu.sync_copy(idx_hbm.at[s, pl.ds(q * _QB, _QB)], idx_v)
        for r in range(_QB // _GROUP):
            pltpu.async_copy(
                table_hbm.at[idx_v.at[pl.ds(r * _GROUP, _GROUP)]],
                rows_v.at[pl.ds(r * _GROUP, _GROUP), :],
                semg,
            )

    def wait_rows(rows_v, semg):
        pltpu.make_async_copy(
            table_hbm.at[pl.ds(0, _QB)], rows_v, semg).wait()

    def drain_store(first):
        @pl.when(jnp.logical_not(first))
        def _():
            pltpu.make_async_copy(
                outt_v, out_hbm.at[0, :, pl.ds(0, _QB // 128)], sems).wait()

    def transpose_store(t, rows_v, first):
        jg = jbase + t
        s = jg // _NQ
        q = jg % _NQ

        drain_store(first)

        @plsc.parallel_loop(0, _QB // 16, unroll=4)
        def tgroup(g):
            r0 = g * 16
            bc = g // 8
            row_ids = r0 + lax.iota(jnp.int32, 16)
            for f in range(_F):
                vals = plsc.load_gather(
                    rows_v, [row_ids, jnp.full((16,), f, jnp.int32)])
                outt_v[f // 8, bc, f % 8, pl.ds((g % 8) * 16, 16)] = vals

        for fb in range(_F // 8):
            pltpu.async_copy(
                outt_v.at[fb],
                out_hbm.at[s, fb, pl.ds(q * (_QB // 128), _QB // 128)],
                sems,
            )

    fetch(0, idx0, rows0, semg0)

    def pair(i, carry):
        ta = 2 * i
        fetch(ta + 1, idx1, rows1, semg1)
        wait_rows(rows0, semg0)
        transpose_store(ta, rows0, i == 0)

        @pl.when(i < _PER_W // 2 - 1)
        def _():
            fetch(ta + 2, idx0, rows0, semg0)

        wait_rows(rows1, semg1)
        transpose_store(ta + 1, rows1, False)
        return carry

    lax.fori_loop(0, _PER_W // 2, pair, 0)

    fetch(_PER_W - 1, idx0, rows0, semg0)
    wait_rows(rows0, semg0)
    transpose_store(_PER_W - 1, rows0, False)
    drain_store(False)


def kernel(inputs, embedding):
    out5 = _embed_gather(inputs.T, embedding)
    return out5.transpose((2, 4, 0, 1, 3)).reshape(_B, _S, _F)
